# baseline (speedup 1.0000x reference)
import jax
import jax.numpy as jnp
from jax.experimental import pallas as pl
from jax.experimental.pallas import tpu as pltpu


def _gate_kernel(T, E, C, NB, Tb, NW, Tw, Tch):
    def body(x_ref, w_ref, comb_ref, pe_ref, pA_sc, pB_sc, probs_sc, carry_sc):
        i = pl.program_id(0)

        @pl.when(i < NB)
        def _gate_phase():
            xb = x_ref[...]
            lg = jax.lax.dot_general(
                w_ref[...], xb, (((1,), (1,)), ((), ())),
                preferred_element_type=jnp.float32)
            mx = jnp.max(lg, axis=0, keepdims=True)
            ex = jnp.exp(lg - mx)
            den = jnp.sum(ex, axis=0, keepdims=True)
            gates = ex / den
            idx = jax.lax.broadcasted_iota(jnp.int32, (E, Tb), 0)
            m1 = jnp.max(gates, axis=0, keepdims=True)
            t1 = jnp.min(jnp.where(gates == m1, idx, E), axis=0, keepdims=True)
            em1 = idx == t1
            g2 = jnp.where(em1, -1.0, gates)
            m2 = jnp.max(g2, axis=0, keepdims=True)
            t2 = jnp.min(jnp.where(g2 == m2, idx, E), axis=0, keepdims=True)
            em2 = idx == t2
            gs = jnp.maximum(m1 + m2, jnp.finfo(jnp.float32).eps)
            cnt = jnp.concatenate(
                [em1.astype(jnp.int32), em2.astype(jnp.int32)], axis=0)
            c = cnt
            s = 1
            while s < Tb:
                c = c + jnp.concatenate(
                    [jnp.zeros((2 * E, s), jnp.int32), c[:, :Tb - s]], axis=1)
                s *= 2
            carry = jnp.where(i > 0, carry_sc[...], 0)
            g = carry + c - cnt
            carry_sc[...] = carry + c[:, Tb - 1:Tb]
            pA = jnp.where(em1, g[:E, :], -1)
            pB = jnp.where(em2, g[E:, :], -1)
            prbT = gates / gs
            RG = Tb // Tch
            for r in range(RG):
                sl = slice(r * Tch, (r + 1) * Tch)
                pA_sc[pl.ds(i * RG + r, 1)] = pA[:, sl][None]
                pB_sc[pl.ds(i * RG + r, 1)] = pB[:, sl][None]
                probs_sc[pl.ds(i * RG + r, 1)] = prbT[:, sl][None]

        @pl.when(i >= NB)
        def _write_phase():
            bb = i - NB
            R = Tw // Tch
            pA = jnp.concatenate(
                [pA_sc[R * bb + r] for r in range(R)], axis=1)
            pB = jnp.concatenate(
                [pB_sc[R * bb + r] for r in range(R)], axis=1)
            total1 = carry_sc[0:E, :]
            q = pB + total1
            peT = jnp.where(pA >= 0,
                            jnp.where(pA < C, pA, C),
                            jnp.where(jnp.logical_and(pB >= 0, q < C), q, C))
            pe = peT.T
            prb = jnp.concatenate(
                [probs_sc[R * bb + r] for r in range(R)], axis=1).T
            pe_ref[...] = pe
            ci3 = jax.lax.broadcasted_iota(jnp.int32, (Tw, E, C), 2)
            me3 = ci3 == pe[:, :, None]
            comb_ref[...] = jnp.where(me3, prb[:, :, None], 0.0)

    return body


def kernel(hidden_states, W):
    b, s, h = hidden_states.shape
    T = b * s
    E = W.shape[0]
    K = 2
    C = max(K, K * T // E)
    NB = 4
    Tb = T // NB
    NW = 8
    Tw = T // NW
    Tch = min(Tb, Tw)
    x = hidden_states.reshape(T, h).astype(jnp.float32)
    w = W.astype(jnp.float32)

    comb, pe = pl.pallas_call(
        _gate_kernel(T, E, C, NB, Tb, NW, Tw, Tch),
        grid=(NB + NW,),
        in_specs=[
            pl.BlockSpec((Tb, h), lambda i: (jnp.minimum(i, NB - 1), 0)),
            pl.BlockSpec((E, h), lambda i: (0, 0)),
        ],
        out_specs=[
            pl.BlockSpec((Tw, E, C), lambda i: (jnp.maximum(i - NB, 0), 0, 0)),
            pl.BlockSpec((Tw, E), lambda i: (jnp.maximum(i - NB, 0), 0)),
        ],
        out_shape=[
            jax.ShapeDtypeStruct((T, E, C), jnp.float32),
            jax.ShapeDtypeStruct((T, E), jnp.int32),
        ],
        scratch_shapes=[
            pltpu.VMEM((T // Tch, E, Tch), jnp.int32),
            pltpu.VMEM((T // Tch, E, Tch), jnp.int32),
            pltpu.VMEM((T // Tch, E, Tch), jnp.float32),
            pltpu.VMEM((2 * E, 1), jnp.int32),
        ],
        compiler_params=pltpu.CompilerParams(
            dimension_semantics=("arbitrary",),
        ),
    )(x, w)
    disp = pe[:, :, None] == jax.lax.broadcasted_iota(jnp.int32, (1, 1, C), 2)
    return comb, disp


# speedup vs baseline: 1.0126x; 1.0126x over previous
import jax
import jax.numpy as jnp
from jax.experimental import pallas as pl
from jax.experimental.pallas import tpu as pltpu


def _gate_kernel(T, E, C, NB, Tb, NW, Tw, Tch):
    def body(x_ref, w_ref, comb_ref, pe_ref, pA_sc, pB_sc, probs_sc, carry_sc):
        i = pl.program_id(0)

        @pl.when(i < NB)
        def _gate_phase():
            xb = x_ref[...]
            lg = jax.lax.dot_general(
                w_ref[...], xb, (((1,), (1,)), ((), ())),
                preferred_element_type=jnp.float32)
            mx = jnp.max(lg, axis=0, keepdims=True)
            ex = jnp.exp(lg - mx)
            den = jnp.sum(ex, axis=0, keepdims=True)
            gates = ex / den
            idx = jax.lax.broadcasted_iota(jnp.int32, (E, Tb), 0)
            m1 = jnp.max(gates, axis=0, keepdims=True)
            t1 = jnp.min(jnp.where(gates == m1, idx, E), axis=0, keepdims=True)
            em1 = idx == t1
            g2 = jnp.where(em1, -1.0, gates)
            m2 = jnp.max(g2, axis=0, keepdims=True)
            t2 = jnp.min(jnp.where(g2 == m2, idx, E), axis=0, keepdims=True)
            em2 = idx == t2
            gs = jnp.maximum(m1 + m2, jnp.finfo(jnp.float32).eps)
            cnt = jnp.concatenate(
                [em1.astype(jnp.int32), em2.astype(jnp.int32)], axis=0)
            c = cnt
            s = 1
            while s < Tb:
                c = c + jnp.concatenate(
                    [jnp.zeros((2 * E, s), jnp.int32), c[:, :Tb - s]], axis=1)
                s *= 2
            carry = jnp.where(i > 0, carry_sc[...], 0)
            g = carry + c - cnt
            carry_sc[...] = carry + c[:, Tb - 1:Tb]
            pA = jnp.where(em1, g[:E, :], -1)
            pB = jnp.where(em2, g[E:, :], -1)
            prbT = gates / gs
            RG = Tb // Tch
            for r in range(RG):
                sl = slice(r * Tch, (r + 1) * Tch)
                pA_sc[pl.ds(i * RG + r, 1)] = pA[:, sl][None]
                pB_sc[pl.ds(i * RG + r, 1)] = pB[:, sl][None]
                probs_sc[pl.ds(i * RG + r, 1)] = prbT[:, sl][None]

        @pl.when(i >= NB)
        def _write_phase():
            bb = i - NB
            R = Tw // Tch
            pA = jnp.concatenate(
                [pA_sc[R * bb + r] for r in range(R)], axis=1)
            pB = jnp.concatenate(
                [pB_sc[R * bb + r] for r in range(R)], axis=1)
            total1 = carry_sc[0:E, :]
            q = pB + total1
            peT = jnp.where(pA >= 0,
                            jnp.where(pA < C, pA, C),
                            jnp.where(jnp.logical_and(pB >= 0, q < C), q, C))
            pe = peT.T
            prb = jnp.concatenate(
                [probs_sc[R * bb + r] for r in range(R)], axis=1).T
            pe_ref[...] = pe
            ci3 = jax.lax.broadcasted_iota(jnp.int32, (Tw, E, C), 2)
            me3 = ci3 == pe[:, :, None]
            comb_ref[...] = jnp.where(me3, prb[:, :, None], 0.0)

    return body


def kernel(hidden_states, W):
    b, s, h = hidden_states.shape
    T = b * s
    E = W.shape[0]
    K = 2
    C = max(K, K * T // E)
    NB = 4
    Tb = T // NB
    NW = 4
    Tw = T // NW
    Tch = min(Tb, Tw)
    x = hidden_states.reshape(T, h).astype(jnp.float32)
    w = W.astype(jnp.float32)

    comb, pe = pl.pallas_call(
        _gate_kernel(T, E, C, NB, Tb, NW, Tw, Tch),
        grid=(NB + NW,),
        in_specs=[
            pl.BlockSpec((Tb, h), lambda i: (jnp.minimum(i, NB - 1), 0)),
            pl.BlockSpec((E, h), lambda i: (0, 0)),
        ],
        out_specs=[
            pl.BlockSpec((Tw, E, C), lambda i: (jnp.maximum(i - NB, 0), 0, 0)),
            pl.BlockSpec((Tw, E), lambda i: (jnp.maximum(i - NB, 0), 0)),
        ],
        out_shape=[
            jax.ShapeDtypeStruct((T, E, C), jnp.float32),
            jax.ShapeDtypeStruct((T, E), jnp.int32),
        ],
        scratch_shapes=[
            pltpu.VMEM((T // Tch, E, Tch), jnp.int32),
            pltpu.VMEM((T // Tch, E, Tch), jnp.int32),
            pltpu.VMEM((T // Tch, E, Tch), jnp.float32),
            pltpu.VMEM((2 * E, 1), jnp.int32),
        ],
        compiler_params=pltpu.CompilerParams(
            dimension_semantics=("arbitrary",),
        ),
    )(x, w)
    disp = pe[:, :, None] == jax.lax.broadcasted_iota(jnp.int32, (1, 1, C), 2)
    return comb, disp
